# baseline (speedup 1.0000x reference)
import jax
import jax.numpy as jnp
from jax.experimental import pallas as pl


def _copy_body(x_ref, ea_ref, xo_ref, eao_ref):
    xo_ref[...] = x_ref[...]
    eao_ref[...] = ea_ref[...]


def kernel(x, x_lstm, encoded_z_gnss, edge_index, edge_attr,
           node_indexes_related_to_agent, edge_indexes_related_to_agent):
    N, DF = x.shape
    E, DE = edge_attr.shape
    LANES = 128
    ER = (E * DE) // LANES
    ea = edge_attr.reshape(ER, LANES)
    G = 10
    xn, ean = pl.pallas_call(
        _copy_body,
        grid=(G,),
        in_specs=[
            pl.BlockSpec((N // G, DF), lambda i: (i, 0)),
            pl.BlockSpec((ER // G, LANES), lambda i: (i, 0)),
        ],
        out_specs=[
            pl.BlockSpec((N // G, DF), lambda i: (i, 0)),
            pl.BlockSpec((ER // G, LANES), lambda i: (i, 0)),
        ],
        out_shape=[
            jax.ShapeDtypeStruct((N, DF), x.dtype),
            jax.ShapeDtypeStruct((ER, LANES), edge_attr.dtype),
        ],
    )(x, ea)
    return (xn, ean.reshape(E, DE))
